# baseline (speedup 1.0000x reference)
import functools

import jax
import jax.numpy as jnp
from jax import lax
from jax.experimental import pallas as pl
from jax.experimental.pallas import tpu as pltpu
from jax.experimental.pallas import tpu_sc as plsc

B, W, H, C = 8, 224, 224, 96
TASK = H * C
NTASK = B * C
NC, NS, L = 2, 16, 16
NW = NC * NS
BT = 4
BS = B - BT
TPW = BS * C // NW
VPR = H // L
RPC = 2
NCHUNK = C // RPC
NEG_INF = float("-inf")
BIG = 2 ** 30


def _task_argmax(buf, cm):
    lane = lax.iota(jnp.int32, L)
    big = jnp.full((L,), BIG, jnp.int32)

    def chunk_body(c, gacc):
        r0 = c * RPC
        accs = [buf[r0, pl.ds(k * L, L)] for k in range(4)]
        for u in range(RPC):
            for k in range(VPR):
                if u == 0 and k < 4:
                    continue
                accs[k % 4] = jnp.maximum(accs[k % 4],
                                          buf[r0 + u, pl.ds(k * L, L)])
        acc = jnp.maximum(jnp.maximum(accs[0], accs[1]),
                          jnp.maximum(accs[2], accs[3]))
        cm[pl.ds(pl.multiple_of(c * L, 8), L)] = acc
        return jnp.maximum(gacc, acc)
    gacc = lax.fori_loop(0, NCHUNK, chunk_body,
                         jnp.full((L,), NEG_INF, jnp.float32))
    m = jnp.max(gacc)

    def fc_body(i, carry):
        alo, ahi = carry
        c0 = i * 8
        for u in range(8):
            v = cm[pl.ds(pl.multiple_of((c0 + u) * L, 8), L)]
            eq = v == m
            alo = jnp.minimum(alo, jnp.where(eq, c0 + u, BIG))
            ahi = jnp.maximum(ahi, jnp.where(eq, c0 + u, -1))
        return alo, ahi
    alo, ahi = lax.fori_loop(0, NCHUNK // 8, fc_body,
                             (big, jnp.full((L,), -1, jnp.int32)))
    cstar = jnp.min(alo)
    cmax = jnp.max(ahi)

    def scan_chunk(c, carry):
        ra, rb = carry
        for u in range(RPC):
            r = c * RPC + u
            for k in range(VPR):
                v = buf[r, pl.ds(k * L, L)]
                q = (k * L + lane) * C + r
                cand = jnp.where(v == m, q, BIG)
                if k % 2 == 0:
                    ra = jnp.minimum(ra, cand)
                else:
                    rb = jnp.minimum(rb, cand)
        return ra, rb

    def one_chunk():
        ra, rb = scan_chunk(cstar, (big, big))
        return jnp.min(jnp.minimum(ra, rb))

    def all_chunks():
        ra, rb = lax.fori_loop(0, NCHUNK, scan_chunk, (big, big))
        return jnp.min(jnp.minimum(ra, rb))

    idx = lax.cond(cmax == cstar, one_chunk, all_chunks)
    col = (idx % W).astype(jnp.float32)
    row = (idx // W).astype(jnp.float32)
    return col, row


_mesh = plsc.VectorSubcoreMesh(core_axis_name="c", subcore_axis_name="s")


@functools.partial(
    pl.kernel,
    mesh=_mesh,
    out_type=jax.ShapeDtypeStruct((BS * C * 2,), jnp.float32),
    scratch_types=[
        pltpu.VMEM((C, H), jnp.float32),
        pltpu.VMEM((C, H), jnp.float32),
        pltpu.VMEM((NCHUNK * L,), jnp.float32),
        pltpu.VMEM((2 * TPW,), jnp.float32),
        pltpu.SemaphoreType.DMA,
        pltpu.SemaphoreType.DMA,
    ],
    compiler_params=pltpu.CompilerParams(
        needs_layout_passes=False, use_tc_tiling_on_sc=True),
)
def _sc_argmax(xt_hbm, out_hbm, buf0, buf1, cm, outv, sem0, sem1):
    wid = lax.axis_index("c") * NS + lax.axis_index("s")
    t0 = wid * TPW
    lane = lax.iota(jnp.int32, L)

    def start(tg, buf, sem):
        pltpu.async_copy(xt_hbm.at[BT + tg // C, tg % C], buf, sem)

    def wait(buf, sem):
        pltpu.make_async_copy(xt_hbm.at[0, 0], buf, sem).wait()

    start(t0, buf0, sem0)
    start(t0 + 1, buf1, sem1)

    def pair_body(q, carry):
        for s_ in range(2):
            buf = (buf0, buf1)[s_]
            sem = (sem0, sem1)[s_]
            t = 2 * q + s_
            wait(buf, sem)
            col, row = _task_argmax(buf, cm)

            @pl.when(q < TPW // 2 - 1)
            def _():
                start(t0 + t + 2, buf, sem)

            val = jnp.where(lane == 0, col, row)
            plsc.store_scatter(outv, [2 * t + lane], val, mask=lane < 2)
        return carry
    lax.fori_loop(0, TPW // 2, pair_body, jnp.int32(0))

    out_off = pl.multiple_of(t0 * 2, 8)
    pltpu.sync_copy(outv, out_hbm.at[pl.ds(out_off, 2 * TPW)])


JB = 16


def _tc_argmax_body(x_ref, o_ref):
    vals = x_ref[0]
    m = jnp.max(vals, axis=(1, 2), keepdims=True)
    q2 = (lax.broadcasted_iota(jnp.int32, (1, C, H), 2) * C
          + lax.broadcasted_iota(jnp.int32, (1, C, H), 1))
    idx = jnp.min(jnp.where(vals == m, q2, BIG), axis=(1, 2))
    col = (idx % W).astype(jnp.float32)
    row = (idx // W).astype(jnp.float32)
    o_ref[...] = jnp.stack([col, row], axis=1)[None]


_JCELLS = C // JB
_tc_call = pl.pallas_call(
    _tc_argmax_body,
    grid=(BT * _JCELLS,),
    in_specs=[pl.BlockSpec((1, JB, C, H),
                           lambda i: (i // _JCELLS, i % _JCELLS, 0, 0))],
    out_specs=pl.BlockSpec((1, JB, 2), lambda i: (i, 0, 0)),
    out_shape=jax.ShapeDtypeStruct((BT * _JCELLS, JB, 2), jnp.float32),
)


def kernel(x):
    xt = jnp.swapaxes(x, 2, 3)
    out_sc = _sc_argmax(xt)
    out_tc = _tc_call(xt)
    return jnp.concatenate(
        [out_tc.reshape(BT, 2 * C), out_sc.reshape(BS, 2 * C)], axis=0)


# speedup vs baseline: 1.0362x; 1.0362x over previous
import functools

import jax
import jax.numpy as jnp
from jax import lax
from jax.experimental import pallas as pl
from jax.experimental.pallas import tpu as pltpu
from jax.experimental.pallas import tpu_sc as plsc

B, W, H, C = 8, 224, 224, 96
TASK = H * C
NTASK = B * C
NC, NS, L = 2, 16, 16
NW = NC * NS
BT = 4
BS = B - BT
TPW = BS * C // NW
VPR = H // L
RPC = 2
NCHUNK = C // RPC
NEG_INF = float("-inf")
BIG = 2 ** 30


def _task_argmax(buf, cm):
    lane = lax.iota(jnp.int32, L)
    big = jnp.full((L,), BIG, jnp.int32)

    def chunk_body(c, gacc):
        r0 = c * RPC
        accs = [buf[r0, pl.ds(k * L, L)] for k in range(4)]
        for u in range(RPC):
            for k in range(VPR):
                if u == 0 and k < 4:
                    continue
                accs[k % 4] = jnp.maximum(accs[k % 4],
                                          buf[r0 + u, pl.ds(k * L, L)])
        acc = jnp.maximum(jnp.maximum(accs[0], accs[1]),
                          jnp.maximum(accs[2], accs[3]))
        cm[pl.ds(pl.multiple_of(c * L, 8), L)] = acc
        return jnp.maximum(gacc, acc)
    gacc = lax.fori_loop(0, NCHUNK, chunk_body,
                         jnp.full((L,), NEG_INF, jnp.float32))
    m = jnp.max(gacc)

    def fc_body(i, carry):
        alo, ahi = carry
        c0 = i * 8
        for u in range(8):
            v = cm[pl.ds(pl.multiple_of((c0 + u) * L, 8), L)]
            eq = v == m
            alo = jnp.minimum(alo, jnp.where(eq, c0 + u, BIG))
            ahi = jnp.maximum(ahi, jnp.where(eq, c0 + u, -1))
        return alo, ahi
    alo, ahi = lax.fori_loop(0, NCHUNK // 8, fc_body,
                             (big, jnp.full((L,), -1, jnp.int32)))
    cstar = jnp.min(alo)
    cmax = jnp.max(ahi)

    def scan_chunk(c, carry):
        ra, rb = carry
        for u in range(RPC):
            r = c * RPC + u
            for k in range(VPR):
                v = buf[r, pl.ds(k * L, L)]
                q = (k * L + lane) * C + r
                cand = jnp.where(v == m, q, BIG)
                if k % 2 == 0:
                    ra = jnp.minimum(ra, cand)
                else:
                    rb = jnp.minimum(rb, cand)
        return ra, rb

    def one_chunk():
        ra, rb = scan_chunk(cstar, (big, big))
        return jnp.min(jnp.minimum(ra, rb))

    def all_chunks():
        ra, rb = lax.fori_loop(0, NCHUNK, scan_chunk, (big, big))
        return jnp.min(jnp.minimum(ra, rb))

    idx = lax.cond(cmax == cstar, one_chunk, all_chunks)
    col = (idx % W).astype(jnp.float32)
    row = (idx // W).astype(jnp.float32)
    return col, row


_mesh = plsc.VectorSubcoreMesh(core_axis_name="c", subcore_axis_name="s")


@functools.partial(
    pl.kernel,
    mesh=_mesh,
    out_type=jax.ShapeDtypeStruct((BS * C * 2,), jnp.float32),
    scratch_types=[
        pltpu.VMEM((C, H), jnp.float32),
        pltpu.VMEM((C, H), jnp.float32),
        pltpu.VMEM((NCHUNK * L,), jnp.float32),
        pltpu.VMEM((2 * TPW,), jnp.float32),
        pltpu.SemaphoreType.DMA,
        pltpu.SemaphoreType.DMA,
    ],
    compiler_params=pltpu.CompilerParams(
        needs_layout_passes=False, use_tc_tiling_on_sc=True),
)
def _sc_argmax(xt_hbm, out_hbm, buf0, buf1, cm, outv, sem0, sem1):
    wid = lax.axis_index("c") * NS + lax.axis_index("s")
    t0 = wid * TPW
    lane = lax.iota(jnp.int32, L)

    def start(tg, buf, sem):
        pltpu.async_copy(xt_hbm.at[BT + tg // C, tg % C], buf, sem)

    def wait(buf, sem):
        pltpu.make_async_copy(xt_hbm.at[0, 0], buf, sem).wait()

    start(t0, buf0, sem0)
    start(t0 + 1, buf1, sem1)

    def pair_body(q, carry):
        for s_ in range(2):
            buf = (buf0, buf1)[s_]
            sem = (sem0, sem1)[s_]
            t = 2 * q + s_
            wait(buf, sem)
            col, row = _task_argmax(buf, cm)

            @pl.when(q < TPW // 2 - 1)
            def _():
                start(t0 + t + 2, buf, sem)

            val = jnp.where(lane == 0, col, row)
            plsc.store_scatter(outv, [2 * t + lane], val, mask=lane < 2)
        return carry
    lax.fori_loop(0, TPW // 2, pair_body, jnp.int32(0))

    out_off = pl.multiple_of(t0 * 2, 8)
    pltpu.sync_copy(outv, out_hbm.at[pl.ds(out_off, 2 * TPW)])


JB = 32


def _tc_argmax_body(x_ref, o_ref):
    vals = x_ref[0]
    m = jnp.max(vals, axis=(1, 2), keepdims=True)
    q2 = (lax.broadcasted_iota(jnp.int32, (1, C, H), 2) * C
          + lax.broadcasted_iota(jnp.int32, (1, C, H), 1))
    idx = jnp.min(jnp.where(vals == m, q2, BIG), axis=(1, 2))
    col = (idx % W).astype(jnp.float32)
    row = (idx // W).astype(jnp.float32)
    o_ref[...] = jnp.stack([col, row], axis=1)[None]


_JCELLS = C // JB
_tc_call = pl.pallas_call(
    _tc_argmax_body,
    grid=(BT * _JCELLS,),
    in_specs=[pl.BlockSpec((1, JB, C, H),
                           lambda i: (i // _JCELLS, i % _JCELLS, 0, 0))],
    out_specs=pl.BlockSpec((1, JB, 2), lambda i: (i, 0, 0)),
    out_shape=jax.ShapeDtypeStruct((BT * _JCELLS, JB, 2), jnp.float32),
)


def kernel(x):
    xt = jnp.swapaxes(x, 2, 3)
    out_sc = _sc_argmax(xt)
    out_tc = _tc_call(xt)
    return jnp.concatenate(
        [out_tc.reshape(BT, 2 * C), out_sc.reshape(BS, 2 * C)], axis=0)


# speedup vs baseline: 1.0407x; 1.0044x over previous
import functools

import jax
import jax.numpy as jnp
from jax import lax
from jax.experimental import pallas as pl
from jax.experimental.pallas import tpu as pltpu
from jax.experimental.pallas import tpu_sc as plsc

B, W, H, C = 8, 224, 224, 96
TASK = H * C
NTASK = B * C
NC, NS, L = 2, 16, 16
NW = NC * NS
BT = 4
BS = B - BT
TPW = BS * C // NW
VPR = H // L
RPC = 2
NCHUNK = C // RPC
NEG_INF = float("-inf")
BIG = 2 ** 30


def _task_argmax(buf, cm):
    lane = lax.iota(jnp.int32, L)
    big = jnp.full((L,), BIG, jnp.int32)

    def chunk_body(c, gacc):
        r0 = c * RPC
        accs = [buf[r0, pl.ds(k * L, L)] for k in range(4)]
        for u in range(RPC):
            for k in range(VPR):
                if u == 0 and k < 4:
                    continue
                accs[k % 4] = jnp.maximum(accs[k % 4],
                                          buf[r0 + u, pl.ds(k * L, L)])
        acc = jnp.maximum(jnp.maximum(accs[0], accs[1]),
                          jnp.maximum(accs[2], accs[3]))
        cm[pl.ds(pl.multiple_of(c * L, 8), L)] = acc
        return jnp.maximum(gacc, acc)
    gacc = lax.fori_loop(0, NCHUNK, chunk_body,
                         jnp.full((L,), NEG_INF, jnp.float32))
    m = jnp.max(gacc)

    def fc_body(i, carry):
        alo, ahi = carry
        c0 = i * 8
        for u in range(8):
            v = cm[pl.ds(pl.multiple_of((c0 + u) * L, 8), L)]
            eq = v == m
            alo = jnp.minimum(alo, jnp.where(eq, c0 + u, BIG))
            ahi = jnp.maximum(ahi, jnp.where(eq, c0 + u, -1))
        return alo, ahi
    alo, ahi = lax.fori_loop(0, NCHUNK // 8, fc_body,
                             (big, jnp.full((L,), -1, jnp.int32)))
    cstar = jnp.min(alo)
    cmax = jnp.max(ahi)

    def scan_chunk(c, carry):
        ra, rb = carry
        for u in range(RPC):
            r = c * RPC + u
            for k in range(VPR):
                v = buf[r, pl.ds(k * L, L)]
                q = (k * L + lane) * C + r
                cand = jnp.where(v == m, q, BIG)
                if k % 2 == 0:
                    ra = jnp.minimum(ra, cand)
                else:
                    rb = jnp.minimum(rb, cand)
        return ra, rb

    def one_chunk():
        ra, rb = scan_chunk(cstar, (big, big))
        return jnp.min(jnp.minimum(ra, rb))

    def all_chunks():
        ra, rb = lax.fori_loop(0, NCHUNK, scan_chunk, (big, big))
        return jnp.min(jnp.minimum(ra, rb))

    idx = lax.cond(cmax == cstar, one_chunk, all_chunks)
    col = (idx % W).astype(jnp.float32)
    row = (idx // W).astype(jnp.float32)
    return col, row


_mesh = plsc.VectorSubcoreMesh(core_axis_name="c", subcore_axis_name="s")


@functools.partial(
    pl.kernel,
    mesh=_mesh,
    out_type=jax.ShapeDtypeStruct((BS * C * 2,), jnp.float32),
    scratch_types=[
        pltpu.VMEM((C, H), jnp.float32),
        pltpu.VMEM((C, H), jnp.float32),
        pltpu.VMEM((NCHUNK * L,), jnp.float32),
        pltpu.VMEM((2 * TPW,), jnp.float32),
        pltpu.SemaphoreType.DMA,
        pltpu.SemaphoreType.DMA,
    ],
    compiler_params=pltpu.CompilerParams(
        needs_layout_passes=False, use_tc_tiling_on_sc=True),
)
def _sc_argmax(xt_hbm, out_hbm, buf0, buf1, cm, outv, sem0, sem1):
    wid = lax.axis_index("c") * NS + lax.axis_index("s")
    t0 = wid * TPW
    lane = lax.iota(jnp.int32, L)

    def start(tg, buf, sem):
        pltpu.async_copy(xt_hbm.at[BT + tg // C, tg % C], buf, sem)

    def wait(buf, sem):
        pltpu.make_async_copy(xt_hbm.at[0, 0], buf, sem).wait()

    start(t0, buf0, sem0)
    start(t0 + 1, buf1, sem1)

    def pair_body(q, carry):
        for s_ in range(2):
            buf = (buf0, buf1)[s_]
            sem = (sem0, sem1)[s_]
            t = 2 * q + s_
            wait(buf, sem)
            col, row = _task_argmax(buf, cm)

            @pl.when(q < TPW // 2 - 1)
            def _():
                start(t0 + t + 2, buf, sem)

            val = jnp.where(lane == 0, col, row)
            plsc.store_scatter(outv, [2 * t + lane], val, mask=lane < 2)
        return carry
    lax.fori_loop(0, TPW // 2, pair_body, jnp.int32(0))

    out_off = pl.multiple_of(t0 * 2, 8)
    pltpu.sync_copy(outv, out_hbm.at[pl.ds(out_off, 2 * TPW)])


JB = 48


def _tc_argmax_body(x_ref, o_ref):
    vals = x_ref[0]
    m = jnp.max(vals, axis=(1, 2), keepdims=True)
    q2 = (lax.broadcasted_iota(jnp.int32, (1, C, H), 2) * C
          + lax.broadcasted_iota(jnp.int32, (1, C, H), 1))
    idx = jnp.min(jnp.where(vals == m, q2, BIG), axis=(1, 2))
    col = (idx % W).astype(jnp.float32)
    row = (idx // W).astype(jnp.float32)
    o_ref[...] = jnp.stack([col, row], axis=1)[None]


_JCELLS = C // JB
_tc_call = pl.pallas_call(
    _tc_argmax_body,
    grid=(BT * _JCELLS,),
    in_specs=[pl.BlockSpec((1, JB, C, H),
                           lambda i: (i // _JCELLS, i % _JCELLS, 0, 0))],
    out_specs=pl.BlockSpec((1, JB, 2), lambda i: (i, 0, 0)),
    out_shape=jax.ShapeDtypeStruct((BT * _JCELLS, JB, 2), jnp.float32),
)


def kernel(x):
    xt = jnp.swapaxes(x, 2, 3)
    out_sc = _sc_argmax(xt)
    out_tc = _tc_call(xt)
    return jnp.concatenate(
        [out_tc.reshape(BT, 2 * C), out_sc.reshape(BS, 2 * C)], axis=0)
